# Initial kernel scaffold; baseline (speedup 1.0000x reference)
import jax
import jax.numpy as jnp
from jax.experimental import pallas as pl


def kernel(memory, last_update, unique_node_ids, unique_messages, timestamps):
    raise NotImplementedError("write your pallas kernel here")


# speedup vs baseline: 1.7743x; 1.7743x over previous
import functools

import jax
import jax.numpy as jnp
from jax import lax
from jax.experimental import pallas as pl
from jax.experimental.pallas import tpu as pltpu
from jax.experimental.pallas import tpu_sc as plsc

_M = 1000000
_D = 64
_B = 16384
_LAMB = 10.0
_L = 16
_NC = 2
_NS = 16
_NW = _NC * _NS
_CHUNK = 128
_CPW = _B // (_NW * _CHUNK)

_mesh = plsc.VectorSubcoreMesh(core_axis_name="c", subcore_axis_name="s")

_SPLAT_DNUMS = lax.GatherDimensionNumbers(
    offset_dims=(), collapsed_slice_dims=(0,), start_index_map=(0,))


def _splat(vec, lane):
    idx = jnp.full((_L, 1), lane, jnp.int32)
    return lax.gather(vec, idx, _SPLAT_DNUMS, (1,),
                      mode=lax.GatherScatterMode.PROMISE_IN_BOUNDS)


@functools.partial(
    pl.kernel,
    out_type=(),
    mesh=_mesh,
    compiler_params=pltpu.CompilerParams(use_tc_tiling_on_sc=False),
    scratch_types=[
        pltpu.VMEM((_CPW, _CHUNK), jnp.int32),
        pltpu.VMEM((_CPW, _CHUNK, _D), jnp.float32),
        pltpu.VMEM((_CPW, _CHUNK, _D), jnp.float32),
        pltpu.VMEM((_CPW, _CHUNK), jnp.float32),
        pltpu.VMEM((_CPW, _CHUNK), jnp.float32),
        pltpu.SemaphoreType.DMA,
        pltpu.SemaphoreType.DMA,
    ],
)
def _sc_update(mem_ref, lu_ref, ids_hbm, msg_hbm, ts_hbm,
               idx_v, msg_v, rows_v, ts_v, lu_v, sem_rows, sem_sc):
    wid = lax.axis_index("s") * _NC + lax.axis_index("c")
    cbase = wid * _CPW
    pltpu.sync_copy(ids_hbm.at[pl.ds(cbase, _CPW)], idx_v)
    pltpu.sync_copy(msg_hbm.at[pl.ds(cbase, _CPW)], msg_v)
    pltpu.sync_copy(ts_hbm.at[pl.ds(cbase, _CPW)], ts_v)
    copies = []
    for j in range(_CPW):
        copies.append(
            pltpu.async_copy(mem_ref.at[idx_v.at[j]], rows_v.at[j], sem_rows))
        copies.append(
            pltpu.async_copy(lu_ref.at[idx_v.at[j]], lu_v.at[j], sem_sc))
    for c in copies:
        c.wait()

    for j in range(_CPW):
        def grp_body(g, carry, j=j):
            r0 = g * _L
            f = jnp.exp((lu_v[j, pl.ds(r0, _L)] - ts_v[j, pl.ds(r0, _L)])
                        * (1.0 / _LAMB))
            for r in range(_L):
                spl = _splat(f, r)
                row = r0 + r
                for c0 in range(0, _D, _L):
                    sl = pl.ds(c0, _L)
                    rows_v[j, row, sl] = (msg_v[j, row, sl]
                                          + spl * rows_v[j, row, sl])
            return carry
        lax.fori_loop(0, _CHUNK // _L, grp_body, 0)

    copies = []
    for j in range(_CPW):
        copies.append(
            pltpu.async_copy(rows_v.at[j], mem_ref.at[idx_v.at[j]], sem_rows))
        copies.append(
            pltpu.async_copy(ts_v.at[j], lu_ref.at[idx_v.at[j]], sem_sc))
    for c in copies:
        c.wait()


def kernel(memory, last_update, unique_node_ids, unique_messages, timestamps):
    ids2 = unique_node_ids.reshape(_NW * _CPW, _CHUNK)
    msg3 = unique_messages.reshape(_NW * _CPW, _CHUNK, _D)
    ts2 = timestamps.reshape(_NW * _CPW, _CHUNK)
    mem_ref = jax.new_ref(memory)
    lu_ref = jax.new_ref(last_update)
    _sc_update(mem_ref, lu_ref, ids2, msg3, ts2)
    return mem_ref[...], lu_ref[...]
